# baseline (speedup 1.0000x reference)
import jax
import jax.numpy as jnp
from jax import lax
from jax.experimental import pallas as pl
from jax.experimental.pallas import tpu as pltpu
from jax.experimental.pallas import tpu_sc as plsc

B = 4096
C = 1000
CP = 1024
NT = CP // 128
BETA = 0.7
LMBDA = 0.5
EPS = 1e-4

NC = 2
NW = 32
RW = B // NW
KG = 64


def _sc_gather_body(idx_hbm, tgt_hbm, out_hbm, idx_v, d_v, sem):
    wid = lax.axis_index("s") * NC + lax.axis_index("c")
    base = wid * RW
    pltpu.sync_copy(idx_hbm.at[pl.ds(base, RW)], idx_v)

    def chunk(ci, carry):
        idx_c = idx_v.at[pl.ds(ci * KG, KG)]
        cps = []
        for t in range(NT):
            off = pl.multiple_of(jnp.full((), t * 128, jnp.int32), 128)
            cps.append(pltpu.async_copy(
                tgt_hbm.at[idx_c, pl.ds(off, 128)],
                d_v.at[:, pl.ds(t * 128, 128)], sem))
        for cp in cps:
            cp.wait()
        pltpu.sync_copy(d_v, out_hbm.at[pl.ds(base + ci * KG, KG)])
        return carry

    lax.fori_loop(0, RW // KG, chunk, 0)


def _gather_stage(index, target):
    mesh = plsc.VectorSubcoreMesh(core_axis_name="c", subcore_axis_name="s")
    f = pl.kernel(
        _sc_gather_body,
        out_type=jax.ShapeDtypeStruct((B, CP), jnp.float32),
        mesh=mesh,
        scratch_types=[
            pltpu.VMEM((RW,), jnp.int32),
            pltpu.VMEM((KG, CP), jnp.float32),
            pltpu.SemaphoreType.DMA,
        ],
        compiler_params=pltpu.CompilerParams(use_tc_tiling_on_sc=True,
                                             needs_layout_passes=False),
    )
    return f(index.astype(jnp.int32), target)


def _fused_body(x_ref, lab_ref, t_ref, out_ref):
    i = pl.program_id(0)
    x = x_ref[...]
    t = t_ref[...]
    lab = lab_ref[0, 0, :]
    m = jnp.max(x, axis=1, keepdims=True)
    e = jnp.exp(x - m)
    z = jnp.sum(e, axis=1, keepdims=True)
    lse = m[:, 0] + jnp.log(z[:, 0])
    p = jnp.clip(e / z, EPS, 1.0 - EPS)
    s = jnp.sum(p, axis=1)
    q = jnp.sum(p * p, axis=1) / s
    g = jnp.sum(t[:, :C] * p, axis=1)
    cols = lax.broadcasted_iota(jnp.int32, x.shape, 1)
    xlab = jnp.sum(jnp.where(cols == lab[:, None], x, 0.0), axis=1)
    ce = lse - xlab
    elr = jnp.log(1.0 - (BETA * g + (1.0 - BETA) * q))
    part = (jnp.sum(ce) + LMBDA * jnp.sum(elr)) * (1.0 / B)

    @pl.when(i == 0)
    def _():
        out_ref[0, 0] = part

    @pl.when(i != 0)
    def _():
        out_ref[0, 0] += part


def _fused_stage(output, label, t_gath):
    nb = 8
    r = B // nb
    lab3 = label.astype(jnp.int32).reshape(nb, 1, r)
    out = pl.pallas_call(
        _fused_body,
        grid=(nb,),
        in_specs=[
            pl.BlockSpec((r, C), lambda i: (i, 0)),
            pl.BlockSpec((1, 1, r), lambda i: (i, 0, 0)),
            pl.BlockSpec((r, CP), lambda i: (i, 0)),
        ],
        out_specs=pl.BlockSpec(memory_space=pltpu.SMEM),
        out_shape=jax.ShapeDtypeStruct((1, 1), jnp.float32),
    )(output, lab3, t_gath)
    return out[0, 0]


def kernel(output, label, index, target):
    t_gath = _gather_stage(index, target)
    return _fused_stage(output, label, t_gath)
